# baseline (speedup 1.0000x reference)
import jax
import jax.numpy as jnp
import numpy as np
from jax.experimental import pallas as pl

EPS = 1e-5


def _bn(x, g, be):
    return g * x / jnp.sqrt(1.0 + EPS) + be


def _apply_mlp(x, layers):
    for l in layers:
        x = jax.nn.relu(_bn(x @ l["W"].T + l["b"], l["g"], l["be"]))
    return x


def _index_points(points, idx):
    batch = jnp.arange(points.shape[0]).reshape((points.shape[0],) + (1,) * (idx.ndim - 1))
    return points[batch, idx]


def _fps(xyz, npoint):
    b, n, _ = xyz.shape
    def step(state, _):
        distance, farthest = state
        centroid = xyz[jnp.arange(b), farthest][:, None, :]
        dist = jnp.sum((xyz - centroid) ** 2, -1)
        distance = jnp.minimum(distance, dist)
        nf = jnp.argmax(distance, -1).astype(jnp.int32)
        return (distance, nf), farthest
    init = (jnp.full((b, n), 1e10, jnp.float32), jnp.zeros((b,), jnp.int32))
    _, cent = jax.lax.scan(step, init, None, length=npoint)
    return jnp.transpose(cent)


def _qbp(radius, nsample, xyz, new_xyz):
    n = xyz.shape[1]
    sqr = jnp.sum((new_xyz[:, :, None, :] - xyz[:, None, :, :]) ** 2, -1)
    idx = jnp.where(sqr > radius ** 2, n, jnp.arange(n)[None, None, :])
    idx = jnp.sort(idx, axis=-1)[:, :, :nsample]
    first = idx[:, :, :1]
    return jnp.where(idx == n, first, idx)


def _sa(xyz, points, npoint, radius, nsample, layers, group_all):
    if group_all:
        new_xyz = jnp.zeros((xyz.shape[0], 1, 3), jnp.float32)
        grouped = xyz[:, None, :, :]
        if points is not None:
            grouped = jnp.concatenate([grouped, points[:, None, :, :]], -1)
    else:
        fps_idx = _fps(xyz, npoint)
        new_xyz = _index_points(xyz, fps_idx)
        idx = _qbp(radius, nsample, xyz, new_xyz)
        grouped = _index_points(xyz, idx) - new_xyz[:, :, None, :]
        if points is not None:
            grouped = jnp.concatenate([grouped, _index_points(points, idx)], -1)
    feat = _apply_mlp(grouped, layers)
    return new_xyz, jnp.max(feat, axis=2)


def _touch_kernel(x_ref, o_ref):
    o_ref[...] = x_ref[...]


def kernel(xyz, params):
    coords = xyz[:, :, :3]
    norm = xyz[:, :, 3:]
    l1_xyz, l1_points = _sa(coords, norm, 512, 0.2, 32, params["sa1"], False)
    l2_xyz, l2_points = _sa(l1_xyz, l1_points, 128, 0.4, 64, params["sa2"], False)
    _, l3_points = _sa(l2_xyz, l2_points, None, None, None, params["sa3"], True)
    l3 = l3_points.reshape(xyz.shape[0], 1024)
    l3 = pl.pallas_call(
        _touch_kernel, out_shape=jax.ShapeDtypeStruct(l3.shape, l3.dtype)
    )(l3)
    h = jax.nn.relu(_bn(l3 @ params["fc1"]["W"].T + params["fc1"]["b"], params["fc1"]["g"], params["fc1"]["be"]))
    feat = jax.nn.relu(_bn(h @ params["fc2"]["W"].T + params["fc2"]["b"], params["fc2"]["g"], params["fc2"]["be"]))
    out = feat @ params["fc3"]["W"].T + params["fc3"]["b"]
    return (out, feat, l3)


# speedup vs baseline: 1.2670x; 1.2670x over previous
import functools

import jax
import jax.numpy as jnp
import numpy as np
from jax.experimental import pallas as pl
from jax.experimental.pallas import tpu as pltpu

EPS = 1e-5
B, N, CIN = 16, 1024, 6


def _eye(n):
    r = jax.lax.broadcasted_iota(jnp.int32, (n, n), 0)
    c = jax.lax.broadcasted_iota(jnp.int32, (n, n), 1)
    return jnp.where(r == c, 1.0, 0.0).astype(jnp.bfloat16)


def _split3(a):
    h = a.astype(jnp.bfloat16)
    r = a - h.astype(jnp.float32)
    m = r.astype(jnp.bfloat16)
    l = (r - m.astype(jnp.float32)).astype(jnp.bfloat16)
    return h, m, l


def _dot_t(a, eye):
    h, m, l = _split3(a)
    d = lambda u: jax.lax.dot_general(u, eye, (((0,), (0,)), ((), ())),
                                      preferred_element_type=jnp.float32)
    return (d(h) + d(m)) + d(l)


def _fps_body(x_ref, y_ref, z_ref,
              n1x_ref, n1y_ref, n1z_ref, n2x_ref, n2y_ref, n2z_ref):
    eye16 = _eye(B)

    def run(X, Y, Z, n, npoint, ox_ref, oy_ref, oz_ref):
        lane = jax.lax.broadcasted_iota(jnp.int32, (B, n), 1)

        def body(i, carry):
            distance, far = carry
            m = (lane == far)
            fm = jnp.where(m, 1.0, 0.0).astype(jnp.float32)
            cx = jnp.sum(X * fm, axis=1, keepdims=True)
            cy = jnp.sum(Y * fm, axis=1, keepdims=True)
            cz = jnp.sum(Z * fm, axis=1, keepdims=True)
            ox_ref[pl.ds(i, 1), :] = _dot_t(cx, eye16)
            oy_ref[pl.ds(i, 1), :] = _dot_t(cy, eye16)
            oz_ref[pl.ds(i, 1), :] = _dot_t(cz, eye16)
            dx = X - cx
            dy = Y - cy
            dz = Z - cz
            dist = (dx * dx + dy * dy) + dz * dz
            distance = jnp.minimum(distance, dist)
            md = jnp.max(distance, axis=1, keepdims=True)
            far = jnp.min(jnp.where(distance == md, lane, n),
                          axis=1, keepdims=True).astype(jnp.int32)
            return distance, far

        init = (jnp.full((B, n), 1e10, jnp.float32),
                jnp.zeros((B, 1), jnp.int32))
        jax.lax.fori_loop(0, npoint, body, init)

    run(x_ref[...], y_ref[...], z_ref[...], N, 512,
        n1x_ref, n1y_ref, n1z_ref)
    eye512 = _eye(512)
    run(_dot_t(n1x_ref[...], eye512), _dot_t(n1y_ref[...], eye512),
        _dot_t(n1z_ref[...], eye512), 512, 128,
        n2x_ref, n2y_ref, n2z_ref)


def _fps_call(x, y, z):
    out = [jax.ShapeDtypeStruct((512, B), jnp.float32)] * 3 + \
          [jax.ShapeDtypeStruct((128, B), jnp.float32)] * 3
    r = pl.pallas_call(_fps_body, out_shape=out)(x, y, z)
    return tuple(a.T for a in r)


def _bq_body(x_ref, y_ref, z_ref, qx_ref, qy_ref, qz_ref, tri_ref, code_ref,
             *, r2, nsample):
    X = x_ref[...].reshape(1, -1)
    Y = y_ref[...].reshape(1, -1)
    Z = z_ref[...].reshape(1, -1)
    qx = qx_ref[...].reshape(-1, 1)
    qy = qy_ref[...].reshape(-1, 1)
    qz = qz_ref[...].reshape(-1, 1)
    dx = qx - X
    dy = qy - Y
    dz = qz - Z
    sqr = (dx * dx + dy * dy) + dz * dz
    mask = sqr <= r2
    mbf = jnp.where(mask, 1.0, 0.0).astype(jnp.bfloat16)
    rank = jax.lax.dot(mbf, tri_ref[...],
                       preferred_element_type=jnp.float32)
    code = jnp.where(mask & (rank <= float(nsample)),
                     rank.astype(jnp.int32), 0)
    code_ref[...] = code[None]


def _bq_call(x, y, z, qx, qy, qz, r2, nsample):
    n = x.shape[1]
    s = qx.shape[1]
    tri = jnp.triu(jnp.ones((n, n), jnp.bfloat16))
    body = functools.partial(_bq_body, r2=r2, nsample=nsample)
    x3 = lambda a: a.reshape(B, 1, a.shape[1])
    return pl.pallas_call(
        body,
        grid=(B,),
        in_specs=[
            pl.BlockSpec((1, 1, n), lambda b: (b, 0, 0)),
            pl.BlockSpec((1, 1, n), lambda b: (b, 0, 0)),
            pl.BlockSpec((1, 1, n), lambda b: (b, 0, 0)),
            pl.BlockSpec((1, 1, s), lambda b: (b, 0, 0)),
            pl.BlockSpec((1, 1, s), lambda b: (b, 0, 0)),
            pl.BlockSpec((1, 1, s), lambda b: (b, 0, 0)),
            pl.BlockSpec((n, n), lambda b: (0, 0)),
        ],
        out_specs=pl.BlockSpec((1, s, n), lambda b: (b, 0, 0)),
        out_shape=jax.ShapeDtypeStruct((B, s, n), jnp.int32),
    )(x3(x), x3(y), x3(z), x3(qx), x3(qy), x3(qz), tri)


def _codes_to_idx(code, nsample):
    ks = jnp.arange(1, nsample + 1, dtype=jnp.int32)
    eq = code[:, :, None, :] == ks[None, None, :, None]
    idx = jnp.argmax(eq, axis=-1).astype(jnp.int32)
    valid = jnp.any(eq, axis=-1)
    first = idx[:, :, :1]
    return jnp.where(valid, idx, first)


def _bn(x, g, be):
    return g * x / jnp.sqrt(1.0 + EPS) + be


def _apply_mlp(x, layers):
    for l in layers:
        x = jax.nn.relu(_bn(x @ l["W"].T + l["b"], l["g"], l["be"]))
    return x


def _index_points(points, idx):
    batch = jnp.arange(points.shape[0]).reshape((points.shape[0],) + (1,) * (idx.ndim - 1))
    return points[batch, idx]


def kernel(xyz, params):
    coords = xyz[:, :, :3]
    norm = xyz[:, :, 3:]
    x, y, z = xyz[..., 0], xyz[..., 1], xyz[..., 2]

    n1x, n1y, n1z, n2x, n2y, n2z = _fps_call(x, y, z)
    new_xyz1 = jnp.stack([n1x, n1y, n1z], axis=-1)
    new_xyz2 = jnp.stack([n2x, n2y, n2z], axis=-1)

    code1 = _bq_call(x, y, z, n1x, n1y, n1z, 0.2 ** 2, 32)
    code2 = _bq_call(n1x, n1y, n1z, n2x, n2y, n2z, 0.4 ** 2, 64)

    idx1 = _codes_to_idx(code1, 32)
    grouped = jnp.concatenate(
        [_index_points(coords, idx1) - new_xyz1[:, :, None, :],
         _index_points(norm, idx1)], -1)
    l1_points = jnp.max(_apply_mlp(grouped, params["sa1"]), axis=2)

    idx2 = _codes_to_idx(code2, 64)
    grouped2 = jnp.concatenate(
        [_index_points(new_xyz1, idx2) - new_xyz2[:, :, None, :],
         _index_points(l1_points, idx2)], -1)
    l2_points = jnp.max(_apply_mlp(grouped2, params["sa2"]), axis=2)

    g3 = jnp.concatenate([new_xyz2, l2_points], -1)[:, None, :, :]
    l3 = jnp.max(_apply_mlp(g3, params["sa3"]), axis=2).reshape(B, 1024)

    h = jax.nn.relu(_bn(l3 @ params["fc1"]["W"].T + params["fc1"]["b"],
                        params["fc1"]["g"], params["fc1"]["be"]))
    feat = jax.nn.relu(_bn(h @ params["fc2"]["W"].T + params["fc2"]["b"],
                           params["fc2"]["g"], params["fc2"]["be"]))
    out = feat @ params["fc3"]["W"].T + params["fc3"]["b"]
    return (out, feat, l3)


# speedup vs baseline: 19.4635x; 15.3620x over previous
import functools

import jax
import jax.numpy as jnp
import numpy as np
from jax.experimental import pallas as pl
from jax.experimental.pallas import tpu as pltpu

EPS = 1e-5
B, N, CIN = 16, 1024, 6


def _eye(n):
    r = jax.lax.broadcasted_iota(jnp.int32, (n, n), 0)
    c = jax.lax.broadcasted_iota(jnp.int32, (n, n), 1)
    return jnp.where(r == c, 1.0, 0.0).astype(jnp.bfloat16)


def _split3(a):
    h = a.astype(jnp.bfloat16)
    r = a - h.astype(jnp.float32)
    m = r.astype(jnp.bfloat16)
    l = (r - m.astype(jnp.float32)).astype(jnp.bfloat16)
    return h, m, l


def _dot_t(a, eye):
    h, m, l = _split3(a)
    d = lambda u: jax.lax.dot_general(u, eye, (((0,), (0,)), ((), ())),
                                      preferred_element_type=jnp.float32)
    return (d(h) + d(m)) + d(l)


def _fps_body(x_ref, y_ref, z_ref,
              n1x_ref, n1y_ref, n1z_ref, n2x_ref, n2y_ref, n2z_ref):
    eye16 = _eye(B)

    def run(X, Y, Z, n, npoint, ox_ref, oy_ref, oz_ref):
        lane = jax.lax.broadcasted_iota(jnp.int32, (B, n), 1)

        def body(i, carry):
            distance, far = carry
            m = (lane == far)
            fm = jnp.where(m, 1.0, 0.0).astype(jnp.float32)
            cx = jnp.sum(X * fm, axis=1, keepdims=True)
            cy = jnp.sum(Y * fm, axis=1, keepdims=True)
            cz = jnp.sum(Z * fm, axis=1, keepdims=True)
            ox_ref[pl.ds(i, 1), :] = _dot_t(cx, eye16)
            oy_ref[pl.ds(i, 1), :] = _dot_t(cy, eye16)
            oz_ref[pl.ds(i, 1), :] = _dot_t(cz, eye16)
            dx = X - cx
            dy = Y - cy
            dz = Z - cz
            dist = (dx * dx + dy * dy) + dz * dz
            distance = jnp.minimum(distance, dist)
            md = jnp.max(distance, axis=1, keepdims=True)
            far = jnp.min(jnp.where(distance == md, lane, n),
                          axis=1, keepdims=True).astype(jnp.int32)
            return distance, far

        init = (jnp.full((B, n), 1e10, jnp.float32),
                jnp.zeros((B, 1), jnp.int32))
        jax.lax.fori_loop(0, npoint, body, init)

    run(x_ref[...], y_ref[...], z_ref[...], N, 512,
        n1x_ref, n1y_ref, n1z_ref)
    eye512 = _eye(512)
    run(_dot_t(n1x_ref[...], eye512), _dot_t(n1y_ref[...], eye512),
        _dot_t(n1z_ref[...], eye512), 512, 128,
        n2x_ref, n2y_ref, n2z_ref)


def _fps_call(x, y, z):
    out = [jax.ShapeDtypeStruct((512, B), jnp.float32)] * 3 + \
          [jax.ShapeDtypeStruct((128, B), jnp.float32)] * 3
    r = pl.pallas_call(_fps_body, out_shape=out)(x, y, z)
    return tuple(a.T for a in r)


def _bq_body(x_ref, y_ref, z_ref, qx_ref, qy_ref, qz_ref, tri_ref, code_ref,
             *, r2, nsample):
    X = x_ref[...].reshape(1, -1)
    Y = y_ref[...].reshape(1, -1)
    Z = z_ref[...].reshape(1, -1)
    qx = qx_ref[...].reshape(-1, 1)
    qy = qy_ref[...].reshape(-1, 1)
    qz = qz_ref[...].reshape(-1, 1)
    dx = qx - X
    dy = qy - Y
    dz = qz - Z
    sqr = (dx * dx + dy * dy) + dz * dz
    mask = sqr <= r2
    mbf = jnp.where(mask, 1.0, 0.0).astype(jnp.bfloat16)
    rank = jax.lax.dot(mbf, tri_ref[...],
                       preferred_element_type=jnp.float32)
    code = jnp.where(mask & (rank <= float(nsample)),
                     rank.astype(jnp.int32), 0)
    code_ref[...] = code[None]


def _bq_call(x, y, z, qx, qy, qz, r2, nsample):
    n = x.shape[1]
    s = qx.shape[1]
    tri = jnp.triu(jnp.ones((n, n), jnp.bfloat16))
    body = functools.partial(_bq_body, r2=r2, nsample=nsample)
    x3 = lambda a: a.reshape(B, 1, a.shape[1])
    return pl.pallas_call(
        body,
        grid=(B,),
        in_specs=[
            pl.BlockSpec((1, 1, n), lambda b: (b, 0, 0)),
            pl.BlockSpec((1, 1, n), lambda b: (b, 0, 0)),
            pl.BlockSpec((1, 1, n), lambda b: (b, 0, 0)),
            pl.BlockSpec((1, 1, s), lambda b: (b, 0, 0)),
            pl.BlockSpec((1, 1, s), lambda b: (b, 0, 0)),
            pl.BlockSpec((1, 1, s), lambda b: (b, 0, 0)),
            pl.BlockSpec((n, n), lambda b: (0, 0)),
        ],
        out_specs=pl.BlockSpec((1, s, n), lambda b: (b, 0, 0)),
        out_shape=jax.ShapeDtypeStruct((B, s, n), jnp.int32),
    )(x3(x), x3(y), x3(z), x3(qx), x3(qy), x3(qz), tri)


def _dot1(a, b):
    return jax.lax.dot(a.astype(jnp.bfloat16), b.astype(jnp.bfloat16),
                       preferred_element_type=jnp.float32)


def _bnrelu(y, b, g, be):
    return jax.nn.relu(g * (y + b) / jnp.sqrt(jnp.float32(1.0 + EPS)) + be)


def _sa_body(pts_ref, q_ref, code_ref,
             w1_ref, b1_ref, g1_ref, e1_ref, w2_ref, b2_ref, g2_ref, e2_ref,
             w3_ref, b3_ref, g3_ref, e3_ref, o_ref, *, nsample):
    K = nsample
    pts = pts_ref[0]
    code = code_ref[0]
    sb = code.shape[0]
    cin = pts.shape[1]
    ph, pm, plo = _split3(pts)

    q = q_ref[0]
    qpad = jnp.concatenate([q, jnp.zeros((sb, cin - 3), jnp.float32)], axis=1)

    d = lambda u, v: jax.lax.dot(u, v, preferred_element_type=jnp.float32)
    gs = []
    for k in range(K):
        oh = jnp.where(code == (k + 1), 1.0, 0.0).astype(jnp.bfloat16)
        p = (d(oh, ph) + d(oh, pm)) + d(oh, plo)
        gs.append(p - qpad)
    G = jnp.concatenate(gs, axis=0)

    h1 = _bnrelu(_dot1(G, w1_ref[...]), b1_ref[...], g1_ref[...], e1_ref[...])
    h2 = _bnrelu(_dot1(h1, w2_ref[...]), b2_ref[...], g2_ref[...], e2_ref[...])
    y3 = _dot1(h2, w3_ref[...]) + b3_ref[...]
    h3 = g3_ref[...] * y3 / jnp.sqrt(jnp.float32(1.0 + EPS)) + e3_ref[...]

    c3 = h3.shape[1]
    cnt = jnp.max(code, axis=1, keepdims=True)
    kio = jax.lax.broadcasted_iota(jnp.int32, (K, sb, 1), 0)
    hm = jnp.where(kio < cnt[None, :, :], h3.reshape(K, sb, c3), -jnp.inf)
    o_ref[0] = jax.nn.relu(jnp.max(hm, axis=0))


def _fold(layer):
    return (layer["W"].T, layer["b"].reshape(1, -1),
            layer["g"].reshape(1, -1), layer["be"].reshape(1, -1))


def _sa_call(pts, q, code, layers, nsample, sblk):
    b, n, cin = pts.shape
    s = q.shape[1]
    c3 = layers[2][0].shape[1]
    body = functools.partial(_sa_body, nsample=nsample)
    wspecs = []
    wargs = []
    for arrs in layers:
        for a in arrs:
            wspecs.append(pl.BlockSpec(a.shape, lambda b_, s_: (0, 0)))
            wargs.append(a)
    return pl.pallas_call(
        body,
        grid=(b, s // sblk),
        in_specs=[
            pl.BlockSpec((1, n, cin), lambda b_, s_: (b_, 0, 0)),
            pl.BlockSpec((1, sblk, 3), lambda b_, s_: (b_, s_, 0)),
            pl.BlockSpec((1, sblk, n), lambda b_, s_: (b_, s_, 0)),
        ] + wspecs,
        out_specs=pl.BlockSpec((1, sblk, c3), lambda b_, s_: (b_, s_, 0)),
        out_shape=jax.ShapeDtypeStruct((b, s, c3), jnp.float32),
    )(pts, q, code, *wargs)


def _head_body(g_ref, w1_ref, b1_ref, g1_ref, e1_ref,
               w2_ref, b2_ref, g2_ref, e2_ref,
               w3_ref, b3_ref, g3_ref, e3_ref,
               f1_ref, fb1_ref, fg1_ref, fe1_ref,
               f2_ref, fb2_ref, fg2_ref, fe2_ref, f3_ref, fb3_ref,
               out_ref, feat_ref, l3_ref):
    h = _bnrelu(_dot1(g_ref[...], w1_ref[...]), b1_ref[...], g1_ref[...], e1_ref[...])
    h = _bnrelu(_dot1(h, w2_ref[...]), b2_ref[...], g2_ref[...], e2_ref[...])
    y = _dot1(h, w3_ref[...]) + b3_ref[...]
    h = g3_ref[...] * y / jnp.sqrt(jnp.float32(1.0 + EPS)) + e3_ref[...]
    l3 = jax.nn.relu(jnp.max(h.reshape(B, 128, 1024), axis=1))
    l3_ref[...] = l3
    h1 = _bnrelu(_dot1(l3, f1_ref[...]), fb1_ref[...], fg1_ref[...], fe1_ref[...])
    feat = _bnrelu(_dot1(h1, f2_ref[...]), fb2_ref[...], fg2_ref[...], fe2_ref[...])
    feat_ref[...] = feat
    out_ref[...] = _dot1(feat, f3_ref[...]) + fb3_ref[...]


def _head_call(g3, sa3, fc1, fc2, w3f, b3f):
    args = [g3]
    for arrs in sa3 + [fc1, fc2]:
        args += list(arrs)
    args += [w3f, b3f]
    out = [jax.ShapeDtypeStruct((B, 2), jnp.float32),
           jax.ShapeDtypeStruct((B, 64), jnp.float32),
           jax.ShapeDtypeStruct((B, 1024), jnp.float32)]
    return pl.pallas_call(_head_body, out_shape=out)(*args)


def kernel(xyz, params):
    x, y, z = xyz[..., 0], xyz[..., 1], xyz[..., 2]

    n1x, n1y, n1z, n2x, n2y, n2z = _fps_call(x, y, z)
    new_xyz1 = jnp.stack([n1x, n1y, n1z], axis=-1)
    new_xyz2 = jnp.stack([n2x, n2y, n2z], axis=-1)

    code1 = _bq_call(x, y, z, n1x, n1y, n1z, 0.2 ** 2, 32)
    code2 = _bq_call(n1x, n1y, n1z, n2x, n2y, n2z, 0.4 ** 2, 64)

    sa1 = [_fold(l) for l in params["sa1"]]
    sa2 = [_fold(l) for l in params["sa2"]]
    sa3 = [_fold(l) for l in params["sa3"]]

    l1_points = _sa_call(xyz, new_xyz1, code1, sa1, 32, 128)
    p2 = jnp.concatenate([new_xyz1, l1_points], -1)
    l2_points = _sa_call(p2, new_xyz2, code2, sa2, 64, 128)

    g3 = jnp.concatenate([new_xyz2, l2_points], -1).reshape(B * 128, 259)
    out, feat, l3 = _head_call(
        g3, sa3, _fold(params["fc1"]), _fold(params["fc2"]),
        params["fc3"]["W"].T, params["fc3"]["b"].reshape(1, -1))
    return (out, feat, l3)


# speedup vs baseline: 19.5282x; 1.0033x over previous
import functools

import jax
import jax.numpy as jnp
import numpy as np
from jax.experimental import pallas as pl
from jax.experimental.pallas import tpu as pltpu

EPS = 1e-5
B, N, CIN = 16, 1024, 6


def _eye(n):
    r = jax.lax.broadcasted_iota(jnp.int32, (n, n), 0)
    c = jax.lax.broadcasted_iota(jnp.int32, (n, n), 1)
    return jnp.where(r == c, 1.0, 0.0).astype(jnp.bfloat16)


def _split3(a):
    h = a.astype(jnp.bfloat16)
    r = a - h.astype(jnp.float32)
    m = r.astype(jnp.bfloat16)
    l = (r - m.astype(jnp.float32)).astype(jnp.bfloat16)
    return h, m, l


def _dot_t(a, eye):
    h, m, l = _split3(a)
    d = lambda u: jax.lax.dot_general(u, eye, (((0,), (0,)), ((), ())),
                                      preferred_element_type=jnp.float32)
    return (d(h) + d(m)) + d(l)


def _fps_body(x_ref, y_ref, z_ref,
              n1x_ref, n1y_ref, n1z_ref, n2x_ref, n2y_ref, n2z_ref):
    def run(X, Y, Z, n, npoint, ox_ref, oy_ref, oz_ref):
        lane = jax.lax.broadcasted_iota(jnp.int32, (B, n), 1)
        olane = jax.lax.broadcasted_iota(jnp.int32, (B, npoint), 1)

        def body(i, carry):
            distance, far, ox, oy, oz = carry
            m = (lane == far)
            fm = jnp.where(m, 1.0, 0.0).astype(jnp.float32)
            cx = jnp.sum(X * fm, axis=1, keepdims=True)
            cy = jnp.sum(Y * fm, axis=1, keepdims=True)
            cz = jnp.sum(Z * fm, axis=1, keepdims=True)
            ox = jnp.where(olane == i, cx, ox)
            oy = jnp.where(olane == i, cy, oy)
            oz = jnp.where(olane == i, cz, oz)
            dx = X - cx
            dy = Y - cy
            dz = Z - cz
            dist = (dx * dx + dy * dy) + dz * dz
            distance = jnp.minimum(distance, dist)
            md = jnp.max(distance, axis=1, keepdims=True)
            far = jnp.min(jnp.where(distance == md, lane, n),
                          axis=1, keepdims=True).astype(jnp.int32)
            return distance, far, ox, oy, oz

        init = (jnp.full((B, n), 1e10, jnp.float32),
                jnp.zeros((B, 1), jnp.int32),
                jnp.zeros((B, npoint), jnp.float32),
                jnp.zeros((B, npoint), jnp.float32),
                jnp.zeros((B, npoint), jnp.float32))
        _, _, ox, oy, oz = jax.lax.fori_loop(0, npoint, body, init)
        ox_ref[...] = ox
        oy_ref[...] = oy
        oz_ref[...] = oz
        return ox, oy, oz

    o1 = run(x_ref[...], y_ref[...], z_ref[...], N, 512,
             n1x_ref, n1y_ref, n1z_ref)
    run(*o1, 512, 128, n2x_ref, n2y_ref, n2z_ref)


def _fps_call(x, y, z):
    out = [jax.ShapeDtypeStruct((B, 512), jnp.float32)] * 3 + \
          [jax.ShapeDtypeStruct((B, 128), jnp.float32)] * 3
    return pl.pallas_call(_fps_body, out_shape=out)(x, y, z)


def _bq_body(x_ref, y_ref, z_ref, qx_ref, qy_ref, qz_ref, tri_ref, code_ref,
             *, r2, nsample):
    X = x_ref[...].reshape(1, -1)
    Y = y_ref[...].reshape(1, -1)
    Z = z_ref[...].reshape(1, -1)
    qx = qx_ref[...].reshape(-1, 1)
    qy = qy_ref[...].reshape(-1, 1)
    qz = qz_ref[...].reshape(-1, 1)
    dx = qx - X
    dy = qy - Y
    dz = qz - Z
    sqr = (dx * dx + dy * dy) + dz * dz
    mask = sqr <= r2
    mbf = jnp.where(mask, 1.0, 0.0).astype(jnp.bfloat16)
    rank = jax.lax.dot(mbf, tri_ref[...],
                       preferred_element_type=jnp.float32)
    code = jnp.where(mask & (rank <= float(nsample)),
                     rank.astype(jnp.int32), 0)
    code_ref[...] = code[None]


def _bq_call(x, y, z, qx, qy, qz, r2, nsample):
    n = x.shape[1]
    s = qx.shape[1]
    tri = jnp.triu(jnp.ones((n, n), jnp.bfloat16))
    body = functools.partial(_bq_body, r2=r2, nsample=nsample)
    x3 = lambda a: a.reshape(B, 1, a.shape[1])
    return pl.pallas_call(
        body,
        grid=(B,),
        in_specs=[
            pl.BlockSpec((1, 1, n), lambda b: (b, 0, 0)),
            pl.BlockSpec((1, 1, n), lambda b: (b, 0, 0)),
            pl.BlockSpec((1, 1, n), lambda b: (b, 0, 0)),
            pl.BlockSpec((1, 1, s), lambda b: (b, 0, 0)),
            pl.BlockSpec((1, 1, s), lambda b: (b, 0, 0)),
            pl.BlockSpec((1, 1, s), lambda b: (b, 0, 0)),
            pl.BlockSpec((n, n), lambda b: (0, 0)),
        ],
        out_specs=pl.BlockSpec((1, s, n), lambda b: (b, 0, 0)),
        out_shape=jax.ShapeDtypeStruct((B, s, n), jnp.int32),
    )(x3(x), x3(y), x3(z), x3(qx), x3(qy), x3(qz), tri)


def _dot1(a, b):
    return jax.lax.dot(a.astype(jnp.bfloat16), b.astype(jnp.bfloat16),
                       preferred_element_type=jnp.float32)


def _bnrelu(y, b, g, be):
    return jax.nn.relu(g * (y + b) / jnp.sqrt(jnp.float32(1.0 + EPS)) + be)


def _sa_body(pts_ref, q_ref, code_ref,
             w1_ref, b1_ref, g1_ref, e1_ref, w2_ref, b2_ref, g2_ref, e2_ref,
             w3_ref, b3_ref, g3_ref, e3_ref, o_ref, *, nsample):
    K = nsample
    pts = pts_ref[0]
    code = code_ref[0]
    sb = code.shape[0]
    cin = pts.shape[1]
    ph, pm, plo = _split3(pts)

    q = q_ref[0]
    qpad = jnp.concatenate([q, jnp.zeros((sb, cin - 3), jnp.float32)], axis=1)

    d = lambda u, v: jax.lax.dot(u, v, preferred_element_type=jnp.float32)
    gs = []
    for k in range(K):
        oh = jnp.where(code == (k + 1), 1.0, 0.0).astype(jnp.bfloat16)
        p = (d(oh, ph) + d(oh, pm)) + d(oh, plo)
        gs.append(p - qpad)
    G = jnp.concatenate(gs, axis=0)

    h1 = _bnrelu(_dot1(G, w1_ref[...]), b1_ref[...], g1_ref[...], e1_ref[...])
    h2 = _bnrelu(_dot1(h1, w2_ref[...]), b2_ref[...], g2_ref[...], e2_ref[...])
    y3 = _dot1(h2, w3_ref[...]) + b3_ref[...]
    h3 = g3_ref[...] * y3 / jnp.sqrt(jnp.float32(1.0 + EPS)) + e3_ref[...]

    c3 = h3.shape[1]
    cnt = jnp.max(code, axis=1, keepdims=True)
    kio = jax.lax.broadcasted_iota(jnp.int32, (K, sb, 1), 0)
    hm = jnp.where(kio < cnt[None, :, :], h3.reshape(K, sb, c3), -jnp.inf)
    o_ref[0] = jax.nn.relu(jnp.max(hm, axis=0))


def _fold(layer):
    return (layer["W"].T, layer["b"].reshape(1, -1),
            layer["g"].reshape(1, -1), layer["be"].reshape(1, -1))


def _sa_call(pts, q, code, layers, nsample, sblk):
    b, n, cin = pts.shape
    s = q.shape[1]
    c3 = layers[2][0].shape[1]
    body = functools.partial(_sa_body, nsample=nsample)
    wspecs = []
    wargs = []
    for arrs in layers:
        for a in arrs:
            wspecs.append(pl.BlockSpec(a.shape, lambda b_, s_: (0, 0)))
            wargs.append(a)
    return pl.pallas_call(
        body,
        grid=(b, s // sblk),
        in_specs=[
            pl.BlockSpec((1, n, cin), lambda b_, s_: (b_, 0, 0)),
            pl.BlockSpec((1, sblk, 3), lambda b_, s_: (b_, s_, 0)),
            pl.BlockSpec((1, sblk, n), lambda b_, s_: (b_, s_, 0)),
        ] + wspecs,
        out_specs=pl.BlockSpec((1, sblk, c3), lambda b_, s_: (b_, s_, 0)),
        out_shape=jax.ShapeDtypeStruct((b, s, c3), jnp.float32),
    )(pts, q, code, *wargs)


def _head_body(g_ref, w1_ref, b1_ref, g1_ref, e1_ref,
               w2_ref, b2_ref, g2_ref, e2_ref,
               w3_ref, b3_ref, g3_ref, e3_ref,
               f1_ref, fb1_ref, fg1_ref, fe1_ref,
               f2_ref, fb2_ref, fg2_ref, fe2_ref, f3_ref, fb3_ref,
               out_ref, feat_ref, l3_ref):
    h = _bnrelu(_dot1(g_ref[...], w1_ref[...]), b1_ref[...], g1_ref[...], e1_ref[...])
    h = _bnrelu(_dot1(h, w2_ref[...]), b2_ref[...], g2_ref[...], e2_ref[...])
    y = _dot1(h, w3_ref[...]) + b3_ref[...]
    h = g3_ref[...] * y / jnp.sqrt(jnp.float32(1.0 + EPS)) + e3_ref[...]
    l3 = jax.nn.relu(jnp.max(h.reshape(B, 128, 1024), axis=1))
    l3_ref[...] = l3
    h1 = _bnrelu(_dot1(l3, f1_ref[...]), fb1_ref[...], fg1_ref[...], fe1_ref[...])
    feat = _bnrelu(_dot1(h1, f2_ref[...]), fb2_ref[...], fg2_ref[...], fe2_ref[...])
    feat_ref[...] = feat
    out_ref[...] = _dot1(feat, f3_ref[...]) + fb3_ref[...]


def _head_call(g3, sa3, fc1, fc2, w3f, b3f):
    args = [g3]
    for arrs in sa3 + [fc1, fc2]:
        args += list(arrs)
    args += [w3f, b3f]
    out = [jax.ShapeDtypeStruct((B, 2), jnp.float32),
           jax.ShapeDtypeStruct((B, 64), jnp.float32),
           jax.ShapeDtypeStruct((B, 1024), jnp.float32)]
    return pl.pallas_call(_head_body, out_shape=out)(*args)


def kernel(xyz, params):
    x, y, z = xyz[..., 0], xyz[..., 1], xyz[..., 2]

    n1x, n1y, n1z, n2x, n2y, n2z = _fps_call(x, y, z)
    new_xyz1 = jnp.stack([n1x, n1y, n1z], axis=-1)
    new_xyz2 = jnp.stack([n2x, n2y, n2z], axis=-1)

    code1 = _bq_call(x, y, z, n1x, n1y, n1z, 0.2 ** 2, 32)
    code2 = _bq_call(n1x, n1y, n1z, n2x, n2y, n2z, 0.4 ** 2, 64)

    sa1 = [_fold(l) for l in params["sa1"]]
    sa2 = [_fold(l) for l in params["sa2"]]
    sa3 = [_fold(l) for l in params["sa3"]]

    l1_points = _sa_call(xyz, new_xyz1, code1, sa1, 32, 128)
    p2 = jnp.concatenate([new_xyz1, l1_points], -1)
    l2_points = _sa_call(p2, new_xyz2, code2, sa2, 64, 128)

    g3 = jnp.concatenate([new_xyz2, l2_points], -1).reshape(B * 128, 259)
    out, feat, l3 = _head_call(
        g3, sa3, _fold(params["fc1"]), _fold(params["fc2"]),
        params["fc3"]["W"].T, params["fc3"]["b"].reshape(1, -1))
    return (out, feat, l3)


# speedup vs baseline: 27.6665x; 1.4168x over previous
import functools

import jax
import jax.numpy as jnp
import numpy as np
from jax.experimental import pallas as pl
from jax.experimental.pallas import tpu as pltpu

EPS = 1e-5
B, N, CIN = 16, 1024, 6


def _eye(n):
    r = jax.lax.broadcasted_iota(jnp.int32, (n, n), 0)
    c = jax.lax.broadcasted_iota(jnp.int32, (n, n), 1)
    return jnp.where(r == c, 1.0, 0.0).astype(jnp.bfloat16)


def _split3(a):
    h = a.astype(jnp.bfloat16)
    r = a - h.astype(jnp.float32)
    m = r.astype(jnp.bfloat16)
    l = (r - m.astype(jnp.float32)).astype(jnp.bfloat16)
    return h, m, l


def _dot_t(a, eye):
    h, m, l = _split3(a)
    d = lambda u: jax.lax.dot_general(u, eye, (((0,), (0,)), ((), ())),
                                      preferred_element_type=jnp.float32)
    return (d(h) + d(m)) + d(l)


def _fps_body(x_ref, y_ref, z_ref,
              n1x_ref, n1y_ref, n1z_ref, n2x_ref, n2y_ref, n2z_ref):
    def run(X, Y, Z, n, npoint, ox_ref, oy_ref, oz_ref):
        lane = jax.lax.broadcasted_iota(jnp.int32, (B, n), 1)
        olane = jax.lax.broadcasted_iota(jnp.int32, (B, npoint), 1)

        def body(i, carry):
            distance, far, ox, oy, oz = carry
            m = (lane == far)
            fm = jnp.where(m, 1.0, 0.0).astype(jnp.float32)
            cx = jnp.sum(X * fm, axis=1, keepdims=True)
            cy = jnp.sum(Y * fm, axis=1, keepdims=True)
            cz = jnp.sum(Z * fm, axis=1, keepdims=True)
            ox = jnp.where(olane == i, cx, ox)
            oy = jnp.where(olane == i, cy, oy)
            oz = jnp.where(olane == i, cz, oz)
            dx = X - cx
            dy = Y - cy
            dz = Z - cz
            dist = (dx * dx + dy * dy) + dz * dz
            distance = jnp.minimum(distance, dist)
            md = jnp.max(distance, axis=1, keepdims=True)
            far = jnp.min(jnp.where(distance == md, lane, n),
                          axis=1, keepdims=True).astype(jnp.int32)
            return distance, far, ox, oy, oz

        init = (jnp.full((B, n), 1e10, jnp.float32),
                jnp.zeros((B, 1), jnp.int32),
                jnp.zeros((B, npoint), jnp.float32),
                jnp.zeros((B, npoint), jnp.float32),
                jnp.zeros((B, npoint), jnp.float32))
        _, _, ox, oy, oz = jax.lax.fori_loop(0, npoint, body, init)
        ox_ref[...] = ox
        oy_ref[...] = oy
        oz_ref[...] = oz
        return ox, oy, oz

    o1 = run(x_ref[...], y_ref[...], z_ref[...], N, 512,
             n1x_ref, n1y_ref, n1z_ref)
    run(*o1, 512, 128, n2x_ref, n2y_ref, n2z_ref)


def _fps_call(x, y, z):
    out = [jax.ShapeDtypeStruct((B, 512), jnp.float32)] * 3 + \
          [jax.ShapeDtypeStruct((B, 128), jnp.float32)] * 3
    return pl.pallas_call(_fps_body, out_shape=out)(x, y, z)


def _bq_body(x_ref, y_ref, z_ref, qx_ref, qy_ref, qz_ref, tri_ref, code_ref,
             *, r2, nsample):
    X = x_ref[...].reshape(1, -1)
    Y = y_ref[...].reshape(1, -1)
    Z = z_ref[...].reshape(1, -1)
    qx = qx_ref[...].reshape(-1, 1)
    qy = qy_ref[...].reshape(-1, 1)
    qz = qz_ref[...].reshape(-1, 1)
    dx = qx - X
    dy = qy - Y
    dz = qz - Z
    sqr = (dx * dx + dy * dy) + dz * dz
    mask = sqr <= r2
    mbf = jnp.where(mask, 1.0, 0.0).astype(jnp.bfloat16)
    rank = jax.lax.dot(mbf, tri_ref[...],
                       preferred_element_type=jnp.float32)
    code = jnp.where(mask & (rank <= float(nsample)),
                     rank.astype(jnp.int32), 0)
    code_ref[...] = code[None]


def _bq_call(x, y, z, qx, qy, qz, r2, nsample):
    n = x.shape[1]
    s = qx.shape[1]
    tri = jnp.triu(jnp.ones((n, n), jnp.bfloat16))
    body = functools.partial(_bq_body, r2=r2, nsample=nsample)
    x3 = lambda a: a.reshape(B, 1, a.shape[1])
    return pl.pallas_call(
        body,
        grid=(B,),
        in_specs=[
            pl.BlockSpec((1, 1, n), lambda b: (b, 0, 0)),
            pl.BlockSpec((1, 1, n), lambda b: (b, 0, 0)),
            pl.BlockSpec((1, 1, n), lambda b: (b, 0, 0)),
            pl.BlockSpec((1, 1, s), lambda b: (b, 0, 0)),
            pl.BlockSpec((1, 1, s), lambda b: (b, 0, 0)),
            pl.BlockSpec((1, 1, s), lambda b: (b, 0, 0)),
            pl.BlockSpec((n, n), lambda b: (0, 0)),
        ],
        out_specs=pl.BlockSpec((1, s, n), lambda b: (b, 0, 0)),
        out_shape=jax.ShapeDtypeStruct((B, s, n), jnp.int32),
    )(x3(x), x3(y), x3(z), x3(qx), x3(qy), x3(qz), tri)


def _dot1(a, b):
    return jax.lax.dot(a.astype(jnp.bfloat16), b.astype(jnp.bfloat16),
                       preferred_element_type=jnp.float32)


def _bnrelu(y, b, g, be):
    return jax.nn.relu(g * (y + b) / jnp.sqrt(jnp.float32(1.0 + EPS)) + be)


def _sa_body(pts_ref, q_ref, code_ref,
             w1_ref, b1_ref, g1_ref, e1_ref, w2_ref, b2_ref, g2_ref, e2_ref,
             w3_ref, b3_ref, g3_ref, e3_ref, o_ref, *, nsample):
    K = nsample
    pts = pts_ref[0]
    code = code_ref[0]
    sb = code.shape[0]
    cin = pts.shape[1]
    ph, pm, plo = _split3(pts)
    pk = jnp.concatenate([ph, pm, plo], axis=1)

    q = q_ref[0]
    qpad = jnp.concatenate([q, jnp.zeros((sb, cin - 3), jnp.float32)], axis=1)

    d = lambda u, v: jax.lax.dot(u, v, preferred_element_type=jnp.float32)
    gs = []
    for k in range(K):
        oh = jnp.where(code == (k + 1), 1.0, 0.0).astype(jnp.bfloat16)
        r = d(oh, pk)
        p = (r[:, 0:cin] + r[:, cin:2 * cin]) + r[:, 2 * cin:3 * cin]
        gs.append(p - qpad)
    G = jnp.concatenate(gs, axis=0)

    h1 = _bnrelu(_dot1(G, w1_ref[...]), b1_ref[...], g1_ref[...], e1_ref[...])
    h2 = _bnrelu(_dot1(h1, w2_ref[...]), b2_ref[...], g2_ref[...], e2_ref[...])
    y3 = _dot1(h2, w3_ref[...]) + b3_ref[...]
    h3 = g3_ref[...] * y3 / jnp.sqrt(jnp.float32(1.0 + EPS)) + e3_ref[...]

    c3 = h3.shape[1]
    cnt = jnp.max(code, axis=1, keepdims=True)
    kio = jax.lax.broadcasted_iota(jnp.int32, (K, sb, 1), 0)
    hm = jnp.where(kio < cnt[None, :, :], h3.reshape(K, sb, c3), -jnp.inf)
    o_ref[0] = jax.nn.relu(jnp.max(hm, axis=0))


def _fold(layer):
    return (layer["W"].T, layer["b"].reshape(1, -1),
            layer["g"].reshape(1, -1), layer["be"].reshape(1, -1))


def _sa_call(pts, q, code, layers, nsample, sblk):
    b, n, cin = pts.shape
    s = q.shape[1]
    c3 = layers[2][0].shape[1]
    body = functools.partial(_sa_body, nsample=nsample)
    wspecs = []
    wargs = []
    for arrs in layers:
        for a in arrs:
            wspecs.append(pl.BlockSpec(a.shape, lambda b_, s_: (0, 0)))
            wargs.append(a)
    return pl.pallas_call(
        body,
        grid=(b, s // sblk),
        in_specs=[
            pl.BlockSpec((1, n, cin), lambda b_, s_: (b_, 0, 0)),
            pl.BlockSpec((1, sblk, 3), lambda b_, s_: (b_, s_, 0)),
            pl.BlockSpec((1, sblk, n), lambda b_, s_: (b_, s_, 0)),
        ] + wspecs,
        out_specs=pl.BlockSpec((1, sblk, c3), lambda b_, s_: (b_, s_, 0)),
        out_shape=jax.ShapeDtypeStruct((b, s, c3), jnp.float32),
    )(pts, q, code, *wargs)


def _head_body(g_ref, w1_ref, b1_ref, g1_ref, e1_ref,
               w2_ref, b2_ref, g2_ref, e2_ref,
               w3_ref, b3_ref, g3_ref, e3_ref,
               f1_ref, fb1_ref, fg1_ref, fe1_ref,
               f2_ref, fb2_ref, fg2_ref, fe2_ref, f3_ref, fb3_ref,
               out_ref, feat_ref, l3_ref):
    h = _bnrelu(_dot1(g_ref[...], w1_ref[...]), b1_ref[...], g1_ref[...], e1_ref[...])
    h = _bnrelu(_dot1(h, w2_ref[...]), b2_ref[...], g2_ref[...], e2_ref[...])
    y = _dot1(h, w3_ref[...]) + b3_ref[...]
    h = g3_ref[...] * y / jnp.sqrt(jnp.float32(1.0 + EPS)) + e3_ref[...]
    l3 = jax.nn.relu(jnp.max(h.reshape(B, 128, 1024), axis=1))
    l3_ref[...] = l3
    h1 = _bnrelu(_dot1(l3, f1_ref[...]), fb1_ref[...], fg1_ref[...], fe1_ref[...])
    feat = _bnrelu(_dot1(h1, f2_ref[...]), fb2_ref[...], fg2_ref[...], fe2_ref[...])
    feat_ref[...] = feat
    out_ref[...] = _dot1(feat, f3_ref[...]) + fb3_ref[...]


def _head_call(g3, sa3, fc1, fc2, w3f, b3f):
    args = [g3]
    for arrs in sa3 + [fc1, fc2]:
        args += list(arrs)
    args += [w3f, b3f]
    out = [jax.ShapeDtypeStruct((B, 2), jnp.float32),
           jax.ShapeDtypeStruct((B, 64), jnp.float32),
           jax.ShapeDtypeStruct((B, 1024), jnp.float32)]
    return pl.pallas_call(_head_body, out_shape=out)(*args)


def kernel(xyz, params):
    x, y, z = xyz[..., 0], xyz[..., 1], xyz[..., 2]

    n1x, n1y, n1z, n2x, n2y, n2z = _fps_call(x, y, z)
    new_xyz1 = jnp.stack([n1x, n1y, n1z], axis=-1)
    new_xyz2 = jnp.stack([n2x, n2y, n2z], axis=-1)

    code1 = _bq_call(x, y, z, n1x, n1y, n1z, 0.2 ** 2, 32)
    code2 = _bq_call(n1x, n1y, n1z, n2x, n2y, n2z, 0.4 ** 2, 64)

    sa1 = [_fold(l) for l in params["sa1"]]
    sa2 = [_fold(l) for l in params["sa2"]]
    sa3 = [_fold(l) for l in params["sa3"]]

    l1_points = _sa_call(xyz, new_xyz1, code1, sa1, 32, 128)
    p2 = jnp.concatenate([new_xyz1, l1_points], -1)
    l2_points = _sa_call(p2, new_xyz2, code2, sa2, 64, 128)

    g3 = jnp.concatenate([new_xyz2, l2_points], -1).reshape(B * 128, 259)
    out, feat, l3 = _head_call(
        g3, sa3, _fold(params["fc1"]), _fold(params["fc2"]),
        params["fc3"]["W"].T, params["fc3"]["b"].reshape(1, -1))
    return (out, feat, l3)


# speedup vs baseline: 28.3965x; 1.0264x over previous
import functools

import jax
import jax.numpy as jnp
import numpy as np
from jax.experimental import pallas as pl
from jax.experimental.pallas import tpu as pltpu

EPS = 1e-5
B, N, CIN = 16, 1024, 6


def _eye(n):
    r = jax.lax.broadcasted_iota(jnp.int32, (n, n), 0)
    c = jax.lax.broadcasted_iota(jnp.int32, (n, n), 1)
    return jnp.where(r == c, 1.0, 0.0).astype(jnp.bfloat16)


def _split3(a):
    h = a.astype(jnp.bfloat16)
    r = a - h.astype(jnp.float32)
    m = r.astype(jnp.bfloat16)
    l = (r - m.astype(jnp.float32)).astype(jnp.bfloat16)
    return h, m, l


def _dot_t(a, eye):
    h, m, l = _split3(a)
    d = lambda u: jax.lax.dot_general(u, eye, (((0,), (0,)), ((), ())),
                                      preferred_element_type=jnp.float32)
    return (d(h) + d(m)) + d(l)


def _fps_body(x_ref, y_ref, z_ref,
              n1x_ref, n1y_ref, n1z_ref, n2x_ref, n2y_ref, n2z_ref):
    def run(X, Y, Z, n, npoint, ox_ref, oy_ref, oz_ref):
        lane = jax.lax.broadcasted_iota(jnp.int32, (B, n), 1)
        olane = jax.lax.broadcasted_iota(jnp.int32, (B, npoint), 1)

        def body(i, carry):
            distance, far, ox, oy, oz = carry
            m = (lane == far)
            fm = jnp.where(m, 1.0, 0.0).astype(jnp.float32)
            cx = jnp.sum(X * fm, axis=1, keepdims=True)
            cy = jnp.sum(Y * fm, axis=1, keepdims=True)
            cz = jnp.sum(Z * fm, axis=1, keepdims=True)
            ox = jnp.where(olane == i, cx, ox)
            oy = jnp.where(olane == i, cy, oy)
            oz = jnp.where(olane == i, cz, oz)
            dx = X - cx
            dy = Y - cy
            dz = Z - cz
            dist = (dx * dx + dy * dy) + dz * dz
            distance = jnp.minimum(distance, dist)
            md = jnp.max(distance, axis=1, keepdims=True)
            far = jnp.min(jnp.where(distance == md, lane, n),
                          axis=1, keepdims=True).astype(jnp.int32)
            return distance, far, ox, oy, oz

        init = (jnp.full((B, n), 1e10, jnp.float32),
                jnp.zeros((B, 1), jnp.int32),
                jnp.zeros((B, npoint), jnp.float32),
                jnp.zeros((B, npoint), jnp.float32),
                jnp.zeros((B, npoint), jnp.float32))
        _, _, ox, oy, oz = jax.lax.fori_loop(0, npoint, body, init)
        ox_ref[...] = ox
        oy_ref[...] = oy
        oz_ref[...] = oz
        return ox, oy, oz

    o1 = run(x_ref[...], y_ref[...], z_ref[...], N, 512,
             n1x_ref, n1y_ref, n1z_ref)
    run(*o1, 512, 128, n2x_ref, n2y_ref, n2z_ref)


def _fps_call(x, y, z):
    out = [jax.ShapeDtypeStruct((B, 512), jnp.float32)] * 3 + \
          [jax.ShapeDtypeStruct((B, 128), jnp.float32)] * 3
    return pl.pallas_call(_fps_body, out_shape=out)(x, y, z)


def _bq_body(x_ref, y_ref, z_ref, qx_ref, qy_ref, qz_ref, tri_ref, code_ref,
             *, r2, nsample):
    X = x_ref[...].reshape(1, -1)
    Y = y_ref[...].reshape(1, -1)
    Z = z_ref[...].reshape(1, -1)
    qx = qx_ref[...].reshape(-1, 1)
    qy = qy_ref[...].reshape(-1, 1)
    qz = qz_ref[...].reshape(-1, 1)
    dx = qx - X
    dy = qy - Y
    dz = qz - Z
    sqr = (dx * dx + dy * dy) + dz * dz
    mask = sqr <= r2
    mbf = jnp.where(mask, 1.0, 0.0).astype(jnp.bfloat16)
    rank = jax.lax.dot(mbf, tri_ref[...],
                       preferred_element_type=jnp.float32)
    code = jnp.where(mask & (rank <= float(nsample)),
                     rank.astype(jnp.int32), 0)
    code_ref[...] = code[None]


def _bq_call(x, y, z, qx, qy, qz, r2, nsample):
    n = x.shape[1]
    s = qx.shape[1]
    tri = jnp.triu(jnp.ones((n, n), jnp.bfloat16))
    body = functools.partial(_bq_body, r2=r2, nsample=nsample)
    x3 = lambda a: a.reshape(B, 1, a.shape[1])
    return pl.pallas_call(
        body,
        grid=(B,),
        in_specs=[
            pl.BlockSpec((1, 1, n), lambda b: (b, 0, 0)),
            pl.BlockSpec((1, 1, n), lambda b: (b, 0, 0)),
            pl.BlockSpec((1, 1, n), lambda b: (b, 0, 0)),
            pl.BlockSpec((1, 1, s), lambda b: (b, 0, 0)),
            pl.BlockSpec((1, 1, s), lambda b: (b, 0, 0)),
            pl.BlockSpec((1, 1, s), lambda b: (b, 0, 0)),
            pl.BlockSpec((n, n), lambda b: (0, 0)),
        ],
        out_specs=pl.BlockSpec((1, s, n), lambda b: (b, 0, 0)),
        out_shape=jax.ShapeDtypeStruct((B, s, n), jnp.int32),
    )(x3(x), x3(y), x3(z), x3(qx), x3(qy), x3(qz), tri)


def _dot1(a, b):
    return jax.lax.dot(a.astype(jnp.bfloat16), b.astype(jnp.bfloat16),
                       preferred_element_type=jnp.float32)


def _bnrelu(y, b, g, be):
    return jax.nn.relu(g * (y + b) / jnp.sqrt(jnp.float32(1.0 + EPS)) + be)


def _sa_body(pts_ref, q_ref, code_ref,
             w1_ref, b1_ref, g1_ref, e1_ref, w2_ref, b2_ref, g2_ref, e2_ref,
             w3_ref, b3_ref, g3_ref, e3_ref, o_ref, *, nsample):
    K = nsample
    pts = pts_ref[0]
    code = code_ref[0]
    sb = code.shape[0]
    cin = pts.shape[1]
    ph, pm, plo = _split3(pts)
    pk = jnp.concatenate([ph, pm, plo], axis=1)

    q = q_ref[0]
    qpad = jnp.concatenate([q, jnp.zeros((sb, cin - 3), jnp.float32)], axis=1)

    d = lambda u, v: jax.lax.dot(u, v, preferred_element_type=jnp.float32)
    gs = []
    for k in range(K):
        oh = jnp.where(code == (k + 1), 1.0, 0.0).astype(jnp.bfloat16)
        r = d(oh, pk)
        p = (r[:, 0:cin] + r[:, cin:2 * cin]) + r[:, 2 * cin:3 * cin]
        gs.append(p - qpad)
    G = jnp.concatenate(gs, axis=0)

    h1 = _bnrelu(_dot1(G, w1_ref[...]), b1_ref[...], g1_ref[...], e1_ref[...])
    h2 = _bnrelu(_dot1(h1, w2_ref[...]), b2_ref[...], g2_ref[...], e2_ref[...])
    y3 = _dot1(h2, w3_ref[...]) + b3_ref[...]
    h3 = g3_ref[...] * y3 / jnp.sqrt(jnp.float32(1.0 + EPS)) + e3_ref[...]

    c3 = h3.shape[1]
    cnt = jnp.max(code, axis=1, keepdims=True)
    kio = jax.lax.broadcasted_iota(jnp.int32, (K, sb, 1), 0)
    hm = jnp.where(kio < cnt[None, :, :], h3.reshape(K, sb, c3), -jnp.inf)
    o_ref[0] = jax.nn.relu(jnp.max(hm, axis=0))


def _fold(layer):
    return (layer["W"].T, layer["b"].reshape(1, -1),
            layer["g"].reshape(1, -1), layer["be"].reshape(1, -1))


def _sa_call(pts, q, code, layers, nsample, sblk):
    b, n, cin = pts.shape
    s = q.shape[1]
    c3 = layers[2][0].shape[1]
    body = functools.partial(_sa_body, nsample=nsample)
    wspecs = []
    wargs = []
    for arrs in layers:
        for a in arrs:
            wspecs.append(pl.BlockSpec(a.shape, lambda b_, s_: (0, 0)))
            wargs.append(a)
    return pl.pallas_call(
        body,
        grid=(b, s // sblk),
        in_specs=[
            pl.BlockSpec((1, n, cin), lambda b_, s_: (b_, 0, 0)),
            pl.BlockSpec((1, sblk, 3), lambda b_, s_: (b_, s_, 0)),
            pl.BlockSpec((1, sblk, n), lambda b_, s_: (b_, s_, 0)),
        ] + wspecs,
        out_specs=pl.BlockSpec((1, sblk, c3), lambda b_, s_: (b_, s_, 0)),
        out_shape=jax.ShapeDtypeStruct((b, s, c3), jnp.float32),
    )(pts, q, code, *wargs)


def _head_body(g_ref, w1_ref, b1_ref, g1_ref, e1_ref,
               w2_ref, b2_ref, g2_ref, e2_ref,
               w3_ref, b3_ref, g3_ref, e3_ref,
               f1_ref, fb1_ref, fg1_ref, fe1_ref,
               f2_ref, fb2_ref, fg2_ref, fe2_ref, f3_ref, fb3_ref,
               out_ref, feat_ref, l3_ref):
    h = _bnrelu(_dot1(g_ref[...], w1_ref[...]), b1_ref[...], g1_ref[...], e1_ref[...])
    h = _bnrelu(_dot1(h, w2_ref[...]), b2_ref[...], g2_ref[...], e2_ref[...])
    y = _dot1(h, w3_ref[...]) + b3_ref[...]
    h = g3_ref[...] * y / jnp.sqrt(jnp.float32(1.0 + EPS)) + e3_ref[...]
    l3 = jax.nn.relu(jnp.max(h.reshape(B, 128, 1024), axis=1))
    l3_ref[...] = l3
    h1 = _bnrelu(_dot1(l3, f1_ref[...]), fb1_ref[...], fg1_ref[...], fe1_ref[...])
    feat = _bnrelu(_dot1(h1, f2_ref[...]), fb2_ref[...], fg2_ref[...], fe2_ref[...])
    feat_ref[...] = feat
    out_ref[...] = _dot1(feat, f3_ref[...]) + fb3_ref[...]


def _head_call(g3, sa3, fc1, fc2, w3f, b3f):
    args = [g3]
    for arrs in sa3 + [fc1, fc2]:
        args += list(arrs)
    args += [w3f, b3f]
    out = [jax.ShapeDtypeStruct((B, 2), jnp.float32),
           jax.ShapeDtypeStruct((B, 64), jnp.float32),
           jax.ShapeDtypeStruct((B, 1024), jnp.float32)]
    return pl.pallas_call(_head_body, out_shape=out)(*args)


def kernel(xyz, params):
    x, y, z = xyz[..., 0], xyz[..., 1], xyz[..., 2]

    n1x, n1y, n1z, n2x, n2y, n2z = _fps_call(x, y, z)
    new_xyz1 = jnp.stack([n1x, n1y, n1z], axis=-1)
    new_xyz2 = jnp.stack([n2x, n2y, n2z], axis=-1)

    code1 = _bq_call(x, y, z, n1x, n1y, n1z, 0.2 ** 2, 32)
    code2 = _bq_call(n1x, n1y, n1z, n2x, n2y, n2z, 0.4 ** 2, 64)

    sa1 = [_fold(l) for l in params["sa1"]]
    sa2 = [_fold(l) for l in params["sa2"]]
    sa3 = [_fold(l) for l in params["sa3"]]

    l1_points = _sa_call(xyz, new_xyz1, code1, sa1, 32, 512)
    p2 = jnp.concatenate([new_xyz1, l1_points], -1)
    l2_points = _sa_call(p2, new_xyz2, code2, sa2, 64, 128)

    g3 = jnp.concatenate([new_xyz2, l2_points], -1).reshape(B * 128, 259)
    out, feat, l3 = _head_call(
        g3, sa3, _fold(params["fc1"]), _fold(params["fc2"]),
        params["fc3"]["W"].T, params["fc3"]["b"].reshape(1, -1))
    return (out, feat, l3)


# speedup vs baseline: 28.5224x; 1.0044x over previous
import functools

import jax
import jax.numpy as jnp
import numpy as np
from jax.experimental import pallas as pl
from jax.experimental.pallas import tpu as pltpu

EPS = 1e-5
B, N, CIN = 16, 1024, 6


def _eye(n):
    r = jax.lax.broadcasted_iota(jnp.int32, (n, n), 0)
    c = jax.lax.broadcasted_iota(jnp.int32, (n, n), 1)
    return jnp.where(r == c, 1.0, 0.0).astype(jnp.bfloat16)


def _split3(a):
    h = a.astype(jnp.bfloat16)
    r = a - h.astype(jnp.float32)
    m = r.astype(jnp.bfloat16)
    l = (r - m.astype(jnp.float32)).astype(jnp.bfloat16)
    return h, m, l


def _dot_t(a, eye):
    h, m, l = _split3(a)
    d = lambda u: jax.lax.dot_general(u, eye, (((0,), (0,)), ((), ())),
                                      preferred_element_type=jnp.float32)
    return (d(h) + d(m)) + d(l)


def _fps_body(x_ref, y_ref, z_ref,
              n1x_ref, n1y_ref, n1z_ref, n2x_ref, n2y_ref, n2z_ref,
              s1x_ref, s1y_ref, s1z_ref, s2x_ref, s2y_ref, s2z_ref):
    def run(X, Y, Z, n, npoint, sx_ref, sy_ref, sz_ref,
            ox_ref, oy_ref, oz_ref):
        lane = jax.lax.broadcasted_iota(jnp.int32, (B, n), 1)

        def body(i, carry):
            distance, far = carry
            m = (lane == far)
            fm = jnp.where(m, 1.0, 0.0).astype(jnp.float32)
            cx = jnp.sum(X * fm, axis=1, keepdims=True)
            cy = jnp.sum(Y * fm, axis=1, keepdims=True)
            cz = jnp.sum(Z * fm, axis=1, keepdims=True)
            sx_ref[pl.ds(i, 1), :] = cx.reshape(1, B)
            sy_ref[pl.ds(i, 1), :] = cy.reshape(1, B)
            sz_ref[pl.ds(i, 1), :] = cz.reshape(1, B)
            dx = X - cx
            dy = Y - cy
            dz = Z - cz
            dist = (dx * dx + dy * dy) + dz * dz
            distance = jnp.minimum(distance, dist)
            md = jnp.max(distance, axis=1, keepdims=True)
            far = jnp.min(jnp.where(distance == md, lane, n),
                          axis=1, keepdims=True).astype(jnp.int32)
            return distance, far

        init = (jnp.full((B, n), 1e10, jnp.float32),
                jnp.zeros((B, 1), jnp.int32))
        jax.lax.fori_loop(0, npoint, body, init)
        eye = _eye(npoint)
        ox = _dot_t(sx_ref[...], eye)
        oy = _dot_t(sy_ref[...], eye)
        oz = _dot_t(sz_ref[...], eye)
        ox_ref[...] = ox
        oy_ref[...] = oy
        oz_ref[...] = oz
        return ox, oy, oz

    o1 = run(x_ref[...], y_ref[...], z_ref[...], N, 512,
             s1x_ref, s1y_ref, s1z_ref, n1x_ref, n1y_ref, n1z_ref)
    run(*o1, 512, 128,
        s2x_ref, s2y_ref, s2z_ref, n2x_ref, n2y_ref, n2z_ref)


def _fps_call(x, y, z):
    out = [jax.ShapeDtypeStruct((B, 512), jnp.float32)] * 3 + \
          [jax.ShapeDtypeStruct((B, 128), jnp.float32)] * 3
    scratch = [pltpu.VMEM((512, B), jnp.float32)] * 3 + \
              [pltpu.VMEM((128, B), jnp.float32)] * 3
    return pl.pallas_call(_fps_body, out_shape=out,
                          scratch_shapes=scratch)(x, y, z)


def _bq_body(x_ref, y_ref, z_ref, qx_ref, qy_ref, qz_ref, tri_ref, code_ref,
             *, r2, nsample):
    X = x_ref[...].reshape(1, -1)
    Y = y_ref[...].reshape(1, -1)
    Z = z_ref[...].reshape(1, -1)
    qx = qx_ref[...].reshape(-1, 1)
    qy = qy_ref[...].reshape(-1, 1)
    qz = qz_ref[...].reshape(-1, 1)
    dx = qx - X
    dy = qy - Y
    dz = qz - Z
    sqr = (dx * dx + dy * dy) + dz * dz
    mask = sqr <= r2
    mbf = jnp.where(mask, 1.0, 0.0).astype(jnp.bfloat16)
    rank = jax.lax.dot(mbf, tri_ref[...],
                       preferred_element_type=jnp.float32)
    code = jnp.where(mask & (rank <= float(nsample)),
                     rank.astype(jnp.int32), 0)
    code_ref[...] = code[None]


def _bq_call(x, y, z, qx, qy, qz, r2, nsample):
    n = x.shape[1]
    s = qx.shape[1]
    tri = jnp.triu(jnp.ones((n, n), jnp.bfloat16))
    body = functools.partial(_bq_body, r2=r2, nsample=nsample)
    x3 = lambda a: a.reshape(B, 1, a.shape[1])
    return pl.pallas_call(
        body,
        grid=(B,),
        in_specs=[
            pl.BlockSpec((1, 1, n), lambda b: (b, 0, 0)),
            pl.BlockSpec((1, 1, n), lambda b: (b, 0, 0)),
            pl.BlockSpec((1, 1, n), lambda b: (b, 0, 0)),
            pl.BlockSpec((1, 1, s), lambda b: (b, 0, 0)),
            pl.BlockSpec((1, 1, s), lambda b: (b, 0, 0)),
            pl.BlockSpec((1, 1, s), lambda b: (b, 0, 0)),
            pl.BlockSpec((n, n), lambda b: (0, 0)),
        ],
        out_specs=pl.BlockSpec((1, s, n), lambda b: (b, 0, 0)),
        out_shape=jax.ShapeDtypeStruct((B, s, n), jnp.int32),
    )(x3(x), x3(y), x3(z), x3(qx), x3(qy), x3(qz), tri)


def _dot1(a, b):
    return jax.lax.dot(a.astype(jnp.bfloat16), b.astype(jnp.bfloat16),
                       preferred_element_type=jnp.float32)


def _bnrelu(y, b, g, be):
    return jax.nn.relu(g * (y + b) / jnp.sqrt(jnp.float32(1.0 + EPS)) + be)


def _sa_body(pts_ref, q_ref, code_ref,
             w1_ref, b1_ref, g1_ref, e1_ref, w2_ref, b2_ref, g2_ref, e2_ref,
             w3_ref, b3_ref, g3_ref, e3_ref, o_ref, *, nsample):
    K = nsample
    pts = pts_ref[0]
    code = code_ref[0]
    sb = code.shape[0]
    cin = pts.shape[1]
    ph, pm, plo = _split3(pts)
    pk = jnp.concatenate([ph, pm, plo], axis=1)

    q = q_ref[0]
    qpad = jnp.concatenate([q, jnp.zeros((sb, cin - 3), jnp.float32)], axis=1)

    d = lambda u, v: jax.lax.dot(u, v, preferred_element_type=jnp.float32)
    gs = []
    for k in range(K):
        oh = jnp.where(code == (k + 1), 1.0, 0.0).astype(jnp.bfloat16)
        r = d(oh, pk)
        p = (r[:, 0:cin] + r[:, cin:2 * cin]) + r[:, 2 * cin:3 * cin]
        gs.append(p - qpad)
    G = jnp.concatenate(gs, axis=0)

    h1 = _bnrelu(_dot1(G, w1_ref[...]), b1_ref[...], g1_ref[...], e1_ref[...])
    h2 = _bnrelu(_dot1(h1, w2_ref[...]), b2_ref[...], g2_ref[...], e2_ref[...])
    y3 = _dot1(h2, w3_ref[...]) + b3_ref[...]
    h3 = g3_ref[...] * y3 / jnp.sqrt(jnp.float32(1.0 + EPS)) + e3_ref[...]

    c3 = h3.shape[1]
    cnt = jnp.max(code, axis=1, keepdims=True)
    kio = jax.lax.broadcasted_iota(jnp.int32, (K, sb, 1), 0)
    hm = jnp.where(kio < cnt[None, :, :], h3.reshape(K, sb, c3), -jnp.inf)
    o_ref[0] = jax.nn.relu(jnp.max(hm, axis=0))


def _fold(layer):
    return (layer["W"].T, layer["b"].reshape(1, -1),
            layer["g"].reshape(1, -1), layer["be"].reshape(1, -1))


def _sa_call(pts, q, code, layers, nsample, sblk):
    b, n, cin = pts.shape
    s = q.shape[1]
    c3 = layers[2][0].shape[1]
    body = functools.partial(_sa_body, nsample=nsample)
    wspecs = []
    wargs = []
    for arrs in layers:
        for a in arrs:
            wspecs.append(pl.BlockSpec(a.shape, lambda b_, s_: (0, 0)))
            wargs.append(a)
    return pl.pallas_call(
        body,
        grid=(b, s // sblk),
        in_specs=[
            pl.BlockSpec((1, n, cin), lambda b_, s_: (b_, 0, 0)),
            pl.BlockSpec((1, sblk, 3), lambda b_, s_: (b_, s_, 0)),
            pl.BlockSpec((1, sblk, n), lambda b_, s_: (b_, s_, 0)),
        ] + wspecs,
        out_specs=pl.BlockSpec((1, sblk, c3), lambda b_, s_: (b_, s_, 0)),
        out_shape=jax.ShapeDtypeStruct((b, s, c3), jnp.float32),
    )(pts, q, code, *wargs)


def _head_body(g_ref, w1_ref, b1_ref, g1_ref, e1_ref,
               w2_ref, b2_ref, g2_ref, e2_ref,
               w3_ref, b3_ref, g3_ref, e3_ref,
               f1_ref, fb1_ref, fg1_ref, fe1_ref,
               f2_ref, fb2_ref, fg2_ref, fe2_ref, f3_ref, fb3_ref,
               out_ref, feat_ref, l3_ref):
    h = _bnrelu(_dot1(g_ref[...], w1_ref[...]), b1_ref[...], g1_ref[...], e1_ref[...])
    h = _bnrelu(_dot1(h, w2_ref[...]), b2_ref[...], g2_ref[...], e2_ref[...])
    y = _dot1(h, w3_ref[...]) + b3_ref[...]
    h = g3_ref[...] * y / jnp.sqrt(jnp.float32(1.0 + EPS)) + e3_ref[...]
    l3 = jax.nn.relu(jnp.max(h.reshape(B, 128, 1024), axis=1))
    l3_ref[...] = l3
    h1 = _bnrelu(_dot1(l3, f1_ref[...]), fb1_ref[...], fg1_ref[...], fe1_ref[...])
    feat = _bnrelu(_dot1(h1, f2_ref[...]), fb2_ref[...], fg2_ref[...], fe2_ref[...])
    feat_ref[...] = feat
    out_ref[...] = _dot1(feat, f3_ref[...]) + fb3_ref[...]


def _head_call(g3, sa3, fc1, fc2, w3f, b3f):
    args = [g3]
    for arrs in sa3 + [fc1, fc2]:
        args += list(arrs)
    args += [w3f, b3f]
    out = [jax.ShapeDtypeStruct((B, 2), jnp.float32),
           jax.ShapeDtypeStruct((B, 64), jnp.float32),
           jax.ShapeDtypeStruct((B, 1024), jnp.float32)]
    return pl.pallas_call(_head_body, out_shape=out)(*args)


def kernel(xyz, params):
    x, y, z = xyz[..., 0], xyz[..., 1], xyz[..., 2]

    n1x, n1y, n1z, n2x, n2y, n2z = _fps_call(x, y, z)
    new_xyz1 = jnp.stack([n1x, n1y, n1z], axis=-1)
    new_xyz2 = jnp.stack([n2x, n2y, n2z], axis=-1)

    code1 = _bq_call(x, y, z, n1x, n1y, n1z, 0.2 ** 2, 32)
    code2 = _bq_call(n1x, n1y, n1z, n2x, n2y, n2z, 0.4 ** 2, 64)

    sa1 = [_fold(l) for l in params["sa1"]]
    sa2 = [_fold(l) for l in params["sa2"]]
    sa3 = [_fold(l) for l in params["sa3"]]

    l1_points = _sa_call(xyz, new_xyz1, code1, sa1, 32, 512)
    p2 = jnp.concatenate([new_xyz1, l1_points], -1)
    l2_points = _sa_call(p2, new_xyz2, code2, sa2, 64, 128)

    g3 = jnp.concatenate([new_xyz2, l2_points], -1).reshape(B * 128, 259)
    out, feat, l3 = _head_call(
        g3, sa3, _fold(params["fc1"]), _fold(params["fc2"]),
        params["fc3"]["W"].T, params["fc3"]["b"].reshape(1, -1))
    return (out, feat, l3)
